# baseline (speedup 1.0000x reference)
import functools

import jax
import jax.numpy as jnp
from jax import lax
from jax.experimental import pallas as pl
from jax.experimental.pallas import tpu as pltpu
from jax.experimental.pallas import tpu_sc as plsc

H = 768
EPS = 1e-12
C = 64


def _make_sc_gather(BT, V):
    info = plsc.get_sparse_core_info()
    NC, NS = info.num_cores, info.num_subcores
    NW = NC * NS
    TPW = BT // NW
    NCHUNK = TPW // C

    mesh = plsc.VectorSubcoreMesh(core_axis_name="c", subcore_axis_name="s")

    @functools.partial(
        pl.kernel,
        mesh=mesh,
        out_type=jax.ShapeDtypeStruct((BT, H), jnp.float32),
        scratch_types=[
            pltpu.VMEM((C,), jnp.int32),
            pltpu.VMEM((C,), jnp.int32),
            pltpu.VMEM((C, H), jnp.float32),
            pltpu.VMEM((C, H), jnp.float32),
            pltpu.SemaphoreType.DMA,
            pltpu.SemaphoreType.DMA,
            pltpu.SemaphoreType.DMA,
            pltpu.SemaphoreType.DMA,
        ],
    )
    def k(ids_hbm, word_hbm, out_hbm,
          idx0, idx1, buf0, buf1, gs0, gs1, ss0, ss1):
        wid = lax.axis_index("s") * NC + lax.axis_index("c")
        base = wid * TPW
        idx = (idx0, idx1)
        buf = (buf0, buf1)
        gsem = (gs0, gs1)
        ssem = (ss0, ss1)

        gathers = [None, None]
        scatters = [None, None]
        for c in range(NCHUNK):
            r = c & 1
            if scatters[r] is not None:
                scatters[r].wait()
            tok0 = base + c * C
            pltpu.sync_copy(ids_hbm.at[pl.ds(tok0, C)], idx[r])
            gathers[r] = pltpu.async_copy(word_hbm.at[idx[r]], buf[r], gsem[r])
            o = 1 - r
            if gathers[o] is not None:
                gathers[o].wait()
                otok0 = base + (c - 1) * C
                scatters[o] = pltpu.async_copy(
                    buf[o], out_hbm.at[pl.ds(otok0, C), :], ssem[o])
                gathers[o] = None
        r = (NCHUNK - 1) & 1
        gathers[r].wait()
        tok0 = base + (NCHUNK - 1) * C
        scatters[r] = pltpu.async_copy(
            buf[r], out_hbm.at[pl.ds(tok0, C), :], ssem[r])
        scatters[0].wait()
        scatters[1].wait()

    return k


def _make_tc_ln(BT, S, B, T):
    TOK = 256
    SCH = S // TOK
    grid = (SCH, B)

    def body(g_ref, pos_ref, tt_ref, type_ref, gam_ref, bet_ref, o_ref):
        x = g_ref[...] + pos_ref[...]
        tt = tt_ref[...]
        t0 = type_ref[0, :][None, :]
        t1 = type_ref[1, :][None, :]
        x = x + jnp.where(tt == 0.0, t0, t1)
        mean = jnp.mean(x, axis=-1, keepdims=True)
        c = x - mean
        var = jnp.mean(c * c, axis=-1, keepdims=True)
        inv = lax.rsqrt(var + EPS)
        o_ref[...] = gam_ref[...] * (c * inv) + bet_ref[...]

    return pl.pallas_call(
        body,
        grid=grid,
        in_specs=[
            pl.BlockSpec((TOK, H), lambda sc, b: (b * SCH + sc, 0)),
            pl.BlockSpec((TOK, H), lambda sc, b: (sc, 0)),
            pl.BlockSpec((TOK, 1), lambda sc, b: (b * SCH + sc, 0)),
            pl.BlockSpec((T, H), lambda sc, b: (0, 0)),
            pl.BlockSpec((1, H), lambda sc, b: (0, 0)),
            pl.BlockSpec((1, H), lambda sc, b: (0, 0)),
        ],
        out_specs=pl.BlockSpec((TOK, H), lambda sc, b: (b * SCH + sc, 0)),
        out_shape=jax.ShapeDtypeStruct((BT, H), jnp.float32),
    )


def kernel(input_ids, token_type_ids, word_emb, pos_emb, type_emb, gamma, beta):
    B, S = input_ids.shape
    V = word_emb.shape[0]
    T = type_emb.shape[0]
    BT = B * S
    NSLICE = 2
    BSL = B // NSLICE
    BTS = BT // NSLICE
    ids = input_ids.reshape(BT).astype(jnp.int32)
    tt_f = token_type_ids.reshape(BT, 1).astype(jnp.float32)
    sc = _make_sc_gather(BTS, V)
    tc = _make_tc_ln(BTS, S, BSL, T)
    g1 = gamma.reshape(1, H)
    b1 = beta.reshape(1, H)
    gathered = [sc(lax.dynamic_slice_in_dim(ids, s * BTS, BTS), word_emb)
                for s in range(NSLICE)]
    outs = [tc(gathered[s], pos_emb,
               lax.dynamic_slice_in_dim(tt_f, s * BTS, BTS), type_emb, g1, b1)
            for s in range(NSLICE)]
    return jnp.concatenate(outs, axis=0).reshape(B, S, H)


# speedup vs baseline: 1.3899x; 1.3899x over previous
import functools

import jax
import jax.numpy as jnp
from jax import lax
from jax.experimental import pallas as pl
from jax.experimental.pallas import tpu as pltpu
from jax.experimental.pallas import tpu_sc as plsc

H = 768
EPS = 1e-12
C = 64


def _make_sc_gather(BT, V):
    info = plsc.get_sparse_core_info()
    NC, NS = info.num_cores, info.num_subcores
    NW = NC * NS
    TPW = BT // NW
    NCHUNK = TPW // C

    mesh = plsc.VectorSubcoreMesh(core_axis_name="c", subcore_axis_name="s")

    @functools.partial(
        pl.kernel,
        mesh=mesh,
        out_type=jax.ShapeDtypeStruct((BT, H), jnp.float32),
        scratch_types=[
            pltpu.VMEM((C,), jnp.int32),
            pltpu.VMEM((C,), jnp.int32),
            pltpu.VMEM((C, H), jnp.float32),
            pltpu.VMEM((C, H), jnp.float32),
            pltpu.SemaphoreType.DMA,
            pltpu.SemaphoreType.DMA,
            pltpu.SemaphoreType.DMA,
            pltpu.SemaphoreType.DMA,
        ],
    )
    def k(ids_hbm, word_hbm, out_hbm,
          idx0, idx1, buf0, buf1, gs0, gs1, ss0, ss1):
        wid = lax.axis_index("s") * NC + lax.axis_index("c")
        base = wid * TPW
        idx = (idx0, idx1)
        buf = (buf0, buf1)
        gsem = (gs0, gs1)
        ssem = (ss0, ss1)

        gathers = [None, None]
        scatters = [None, None]
        for c in range(NCHUNK):
            r = c & 1
            if scatters[r] is not None:
                scatters[r].wait()
            tok0 = base + c * C
            pltpu.sync_copy(ids_hbm.at[pl.ds(tok0, C)], idx[r])
            gathers[r] = pltpu.async_copy(word_hbm.at[idx[r]], buf[r], gsem[r])
            o = 1 - r
            if gathers[o] is not None:
                gathers[o].wait()
                otok0 = base + (c - 1) * C
                scatters[o] = pltpu.async_copy(
                    buf[o], out_hbm.at[pl.ds(otok0, C), :], ssem[o])
                gathers[o] = None
        r = (NCHUNK - 1) & 1
        gathers[r].wait()
        tok0 = base + (NCHUNK - 1) * C
        scatters[r] = pltpu.async_copy(
            buf[r], out_hbm.at[pl.ds(tok0, C), :], ssem[r])
        scatters[0].wait()
        scatters[1].wait()

    return k


def _make_tc_ln(BT, S, B, T):
    TOK = 512
    SCH = S // TOK
    grid = (SCH, B)

    def body(g_ref, pos_ref, tt_ref, type_ref, gam_ref, bet_ref, o_ref):
        x = g_ref[...] + pos_ref[...]
        tt = tt_ref[...]
        t0 = type_ref[0, :][None, :]
        t1 = type_ref[1, :][None, :]
        x = x + jnp.where(tt == 0.0, t0, t1)
        mean = jnp.mean(x, axis=-1, keepdims=True)
        c = x - mean
        var = jnp.mean(c * c, axis=-1, keepdims=True)
        inv = lax.rsqrt(var + EPS)
        o_ref[...] = gam_ref[...] * (c * inv) + bet_ref[...]

    return pl.pallas_call(
        body,
        grid=grid,
        in_specs=[
            pl.BlockSpec((TOK, H), lambda sc, b: (b * SCH + sc, 0)),
            pl.BlockSpec((TOK, H), lambda sc, b: (sc, 0)),
            pl.BlockSpec((TOK, 1), lambda sc, b: (b * SCH + sc, 0)),
            pl.BlockSpec((T, H), lambda sc, b: (0, 0)),
            pl.BlockSpec((1, H), lambda sc, b: (0, 0)),
            pl.BlockSpec((1, H), lambda sc, b: (0, 0)),
        ],
        out_specs=pl.BlockSpec((TOK, H), lambda sc, b: (b * SCH + sc, 0)),
        out_shape=jax.ShapeDtypeStruct((BT, H), jnp.float32),
    )


def kernel(input_ids, token_type_ids, word_emb, pos_emb, type_emb, gamma, beta):
    B, S = input_ids.shape
    V = word_emb.shape[0]
    T = type_emb.shape[0]
    BT = B * S
    ids = input_ids.reshape(BT).astype(jnp.int32)
    tt_f = token_type_ids.reshape(BT, 1).astype(jnp.float32)
    gathered = _make_sc_gather(BT, V)(ids, word_emb)
    out = _make_tc_ln(BT, S, B, T)(
        gathered, pos_emb, tt_f, type_emb,
        gamma.reshape(1, H), beta.reshape(1, H))
    return out.reshape(B, S, H)


# speedup vs baseline: 1.4049x; 1.0108x over previous
import functools

import jax
import jax.numpy as jnp
from jax import lax
from jax.experimental import pallas as pl
from jax.experimental.pallas import tpu as pltpu
from jax.experimental.pallas import tpu_sc as plsc

H = 768
EPS = 1e-12
C = 32
NBUF = 4


def _make_sc_gather(BT, V):
    info = plsc.get_sparse_core_info()
    NC, NS = info.num_cores, info.num_subcores
    NW = NC * NS
    TPW = BT // NW
    NCHUNK = TPW // C
    LAG = NBUF - 1

    mesh = plsc.VectorSubcoreMesh(core_axis_name="c", subcore_axis_name="s")

    @functools.partial(
        pl.kernel,
        mesh=mesh,
        out_type=jax.ShapeDtypeStruct((BT, H), jnp.float32),
        scratch_types=[
            pltpu.VMEM((TPW,), jnp.int32),
        ] + [pltpu.VMEM((C, H), jnp.float32) for _ in range(NBUF)]
          + [pltpu.SemaphoreType.DMA for _ in range(2 * NBUF)],
    )
    def k(ids_hbm, word_hbm, out_hbm, idx_all, *rest):
        bufs = rest[:NBUF]
        gsem = rest[NBUF:2 * NBUF]
        ssem = rest[2 * NBUF:]
        wid = lax.axis_index("s") * NC + lax.axis_index("c")
        base = wid * TPW
        pltpu.sync_copy(ids_hbm.at[pl.ds(base, TPW)], idx_all)

        gathers = [None] * NCHUNK
        scatters = [None] * NCHUNK

        def start_scatter(d):
            gathers[d].wait()
            scatters[d] = pltpu.async_copy(
                bufs[d % NBUF], out_hbm.at[pl.ds(base + d * C, C), :],
                ssem[d % NBUF])

        for c in range(NCHUNK):
            r = c % NBUF
            if c >= NBUF:
                scatters[c - NBUF].wait()
            gathers[c] = pltpu.async_copy(
                word_hbm.at[idx_all.at[pl.ds(c * C, C)]], bufs[r], gsem[r])
            if c >= LAG:
                start_scatter(c - LAG)
        for d in range(NCHUNK - LAG, NCHUNK):
            start_scatter(d)
        for d in range(NCHUNK - NBUF, NCHUNK):
            scatters[d].wait()

    return k


def _make_tc_ln(BT, S, B, T):
    TOK = 512
    SCH = S // TOK
    grid = (SCH, B)

    def body(g_ref, pos_ref, tt_ref, type_ref, gam_ref, bet_ref, o_ref):
        x = g_ref[...] + pos_ref[...]
        tt = tt_ref[...]
        t0 = type_ref[0, :][None, :]
        t1 = type_ref[1, :][None, :]
        x = x + jnp.where(tt == 0.0, t0, t1)
        mean = jnp.mean(x, axis=-1, keepdims=True)
        c = x - mean
        var = jnp.mean(c * c, axis=-1, keepdims=True)
        inv = lax.rsqrt(var + EPS)
        o_ref[...] = gam_ref[...] * (c * inv) + bet_ref[...]

    return pl.pallas_call(
        body,
        grid=grid,
        in_specs=[
            pl.BlockSpec((TOK, H), lambda sc, b: (b * SCH + sc, 0)),
            pl.BlockSpec((TOK, H), lambda sc, b: (sc, 0)),
            pl.BlockSpec((TOK, 1), lambda sc, b: (b * SCH + sc, 0)),
            pl.BlockSpec((T, H), lambda sc, b: (0, 0)),
            pl.BlockSpec((1, H), lambda sc, b: (0, 0)),
            pl.BlockSpec((1, H), lambda sc, b: (0, 0)),
        ],
        out_specs=pl.BlockSpec((TOK, H), lambda sc, b: (b * SCH + sc, 0)),
        out_shape=jax.ShapeDtypeStruct((BT, H), jnp.float32),
    )


def kernel(input_ids, token_type_ids, word_emb, pos_emb, type_emb, gamma, beta):
    B, S = input_ids.shape
    V = word_emb.shape[0]
    T = type_emb.shape[0]
    BT = B * S
    ids = input_ids.reshape(BT).astype(jnp.int32)
    tt_f = token_type_ids.reshape(BT, 1).astype(jnp.float32)
    gathered = _make_sc_gather(BT, V)(ids, word_emb)
    out = _make_tc_ln(BT, S, B, T)(
        gathered, pos_emb, tt_f, type_emb,
        gamma.reshape(1, H), beta.reshape(1, H))
    return out.reshape(B, S, H)


# speedup vs baseline: 1.4946x; 1.0638x over previous
import functools

import jax
import jax.numpy as jnp
from jax import lax
from jax.experimental import pallas as pl
from jax.experimental.pallas import tpu as pltpu
from jax.experimental.pallas import tpu_sc as plsc

H = 768
EPS = 1e-12
C = 32
NBUF = 4


def _make_sc_gather(B, S, V):
    info = plsc.get_sparse_core_info()
    NC, NS = info.num_cores, info.num_subcores
    NW = NC * NS
    TPW = (B * S) // NW
    WPR = S // TPW
    NCHUNK = TPW // C
    LAG = NBUF - 1

    mesh = plsc.VectorSubcoreMesh(core_axis_name="c", subcore_axis_name="s")

    @functools.partial(
        pl.kernel,
        mesh=mesh,
        out_type=jax.ShapeDtypeStruct((B, S, H), jnp.float32),
        scratch_types=[
            pltpu.VMEM((TPW,), jnp.int32),
        ] + [pltpu.VMEM((C, H), jnp.float32) for _ in range(NBUF)]
          + [pltpu.SemaphoreType.DMA for _ in range(2 * NBUF)],
    )
    def k(ids_hbm, word_hbm, out_hbm, idx_all, *rest):
        bufs = rest[:NBUF]
        gsem = rest[NBUF:2 * NBUF]
        ssem = rest[2 * NBUF:]
        wid = lax.axis_index("s") * NC + lax.axis_index("c")
        row = wid // WPR
        col0 = (wid % WPR) * TPW
        pltpu.sync_copy(ids_hbm.at[row, pl.ds(col0, TPW)], idx_all)

        gathers = [None] * NCHUNK
        scatters = [None] * NCHUNK

        def start_scatter(d):
            gathers[d].wait()
            scatters[d] = pltpu.async_copy(
                bufs[d % NBUF],
                out_hbm.at[row, pl.ds(col0 + d * C, C), :],
                ssem[d % NBUF])

        for c in range(NCHUNK):
            r = c % NBUF
            if c >= NBUF:
                scatters[c - NBUF].wait()
            gathers[c] = pltpu.async_copy(
                word_hbm.at[idx_all.at[pl.ds(c * C, C)]], bufs[r], gsem[r])
            if c >= LAG:
                start_scatter(c - LAG)
        for d in range(NCHUNK - LAG, NCHUNK):
            start_scatter(d)
        for d in range(NCHUNK - NBUF, NCHUNK):
            scatters[d].wait()

    return k


def _make_tc_ln(BT, S, B, T):
    TOK = 1024
    SCH = S // TOK
    grid = (SCH, B)

    def body(g_ref, pos_ref, tt_ref, type_ref, gam_ref, bet_ref, o_ref):
        x = g_ref[0] + pos_ref[...]
        tt = tt_ref[...]
        t0 = type_ref[0, :][None, :]
        t1 = type_ref[1, :][None, :]
        x = x + jnp.where(tt == 0, t0, t1)
        mean = jnp.mean(x, axis=-1, keepdims=True)
        c = x - mean
        var = jnp.mean(c * c, axis=-1, keepdims=True)
        inv = lax.rsqrt(var + EPS)
        o_ref[0] = gam_ref[...] * (c * inv) + bet_ref[...]

    return pl.pallas_call(
        body,
        grid=grid,
        in_specs=[
            pl.BlockSpec((1, TOK, H), lambda sc, b: (b, sc, 0)),
            pl.BlockSpec((TOK, H), lambda sc, b: (sc, 0)),
            pl.BlockSpec((TOK, 1), lambda sc, b: (b * SCH + sc, 0)),
            pl.BlockSpec((T, H), lambda sc, b: (0, 0)),
            pl.BlockSpec((1, H), lambda sc, b: (0, 0)),
            pl.BlockSpec((1, H), lambda sc, b: (0, 0)),
        ],
        out_specs=pl.BlockSpec((1, TOK, H), lambda sc, b: (b, sc, 0)),
        out_shape=jax.ShapeDtypeStruct((B, S, H), jnp.float32),
    )


def kernel(input_ids, token_type_ids, word_emb, pos_emb, type_emb, gamma, beta):
    B, S = input_ids.shape
    V = word_emb.shape[0]
    T = type_emb.shape[0]
    BT = B * S
    gathered = _make_sc_gather(B, S, V)(input_ids.astype(jnp.int32), word_emb)
    out = _make_tc_ln(BT, S, B, T)(
        gathered, pos_emb,
        token_type_ids.astype(jnp.int32).reshape(BT, 1), type_emb,
        gamma.reshape(1, H), beta.reshape(1, H))
    return out
